# initial kernel scaffold (unmeasured)
import jax
import jax.numpy as jnp
from jax import lax
from jax.experimental import pallas as pl
from jax.experimental.pallas import tpu as pltpu

N_DEV = 4
SQ = 2048
SKV = 2048
H = 8
DH = 128
D = H * DH
BLK = 64
BLOCKS_PER_SHARD = SKV // BLK
KV_TILE = 1024
SCALE = 0.08838834764831843
NEG = -1e9


def _prep_body(x_ref, wq_ref, k_ref, v_ref, q_out, k_out, v_out):
    q = jnp.dot(x_ref[...], wq_ref[...], preferred_element_type=jnp.float32)
    q_out[...] = q.astype(jnp.bfloat16)
    k_out[...] = k_ref[...].astype(jnp.bfloat16)
    v_out[...] = v_ref[...].astype(jnp.bfloat16)


def _attn_body(q_ref, k_ref, v_ref, out_ref, comm_ref, send_sems, recv_sems):
    my = lax.axis_index("i")
    left = lax.rem(my + N_DEV - 1, N_DEV)
    right = lax.rem(my + 1, N_DEV)

    barrier_sem = pltpu.get_barrier_semaphore()
    for nbr in (left, right):
        pl.semaphore_signal(
            barrier_sem, inc=1,
            device_id=(nbr,), device_id_type=pl.DeviceIdType.MESH,
        )
    pl.semaphore_wait(barrier_sem, 2)

    comm_ref[0, 0] = k_ref[...]
    comm_ref[0, 1] = v_ref[...]

    out_ref[...] = jnp.zeros((SQ, D), jnp.float32)

    m_st = [jnp.full((SQ, 1), -1e30, jnp.float32) for _ in range(H)]
    l_st = [jnp.zeros((SQ, 1), jnp.float32) for _ in range(H)]

    row = lax.broadcasted_iota(jnp.int32, (SQ, 1), 0)
    qb = my * BLOCKS_PER_SHARD + row // BLK

    def process(slot, origin):
        for t0 in range(0, SKV, KV_TILE):
            col = lax.broadcasted_iota(jnp.int32, (1, KV_TILE), 1)
            kb = origin * BLOCKS_PER_SHARD + (t0 + col) // BLK
            keep = (qb == kb) | (kb == 0) | ((qb + kb) % 3 == 0)
            bias = jnp.where(keep, 0.0, NEG).astype(jnp.bfloat16)
            for h in range(H):
                hs = slice(h * DH, (h + 1) * DH)
                qh = q_ref[:, hs]
                kh = comm_ref[slot, 0, t0:t0 + KV_TILE, hs]
                vh = comm_ref[slot, 1, t0:t0 + KV_TILE, hs]
                s = lax.dot_general(
                    qh, kh, (((1,), (1,)), ((), ())),
                    preferred_element_type=jnp.float32,
                ) * SCALE + bias
                m_cur = jnp.max(s, axis=1, keepdims=True)
                m_new = jnp.maximum(m_st[h], m_cur)
                alpha = jnp.exp(m_st[h] - m_new)
                p = jnp.exp(s - m_new)
                l_st[h] = l_st[h] * alpha + jnp.sum(p, axis=1, keepdims=True)
                pv = jnp.dot(
                    p.astype(jnp.bfloat16), vh,
                    preferred_element_type=jnp.float32,
                )
                out_ref[:, hs] = out_ref[:, hs] * alpha + pv
                m_st[h] = m_new

    process(0, my)

    for hop in range(N_DEV - 1):
        send_slot = hop % 2
        recv_slot = (hop + 1) % 2
        rdma = pltpu.make_async_remote_copy(
            src_ref=comm_ref.at[send_slot],
            dst_ref=comm_ref.at[recv_slot],
            send_sem=send_sems.at[send_slot],
            recv_sem=recv_sems.at[recv_slot],
            device_id=(right,),
            device_id_type=pl.DeviceIdType.MESH,
        )
        rdma.start()
        rdma.wait()
        process(recv_slot, lax.rem(my - hop - 1 + N_DEV, N_DEV))

    for h in range(H):
        hs = slice(h * DH, (h + 1) * DH)
        out_ref[:, hs] = out_ref[:, hs] / l_st[h]


def _out_body(ctx_ref, wo_ref, out_ref):
    out_ref[...] = jnp.dot(
        ctx_ref[...], wo_ref[...], preferred_element_type=jnp.float32
    )


def kernel(x, Wq, K_ext, V_ext, Wo):
    x2 = x.reshape(SQ, D)
    k2 = K_ext.reshape(SKV, D)
    v2 = V_ext.reshape(SKV, D)

    qb16, kb16, vb16 = pl.pallas_call(
        _prep_body,
        out_shape=(
            jax.ShapeDtypeStruct((SQ, D), jnp.bfloat16),
            jax.ShapeDtypeStruct((SKV, D), jnp.bfloat16),
            jax.ShapeDtypeStruct((SKV, D), jnp.bfloat16),
        ),
        in_specs=[pl.BlockSpec(memory_space=pltpu.VMEM)] * 4,
        out_specs=(pl.BlockSpec(memory_space=pltpu.VMEM),) * 3,
    )(x2, Wq, k2, v2)

    ctx = pl.pallas_call(
        _attn_body,
        out_shape=jax.ShapeDtypeStruct((SQ, D), jnp.float32),
        in_specs=[pl.BlockSpec(memory_space=pltpu.VMEM)] * 3,
        out_specs=pl.BlockSpec(memory_space=pltpu.VMEM),
        scratch_shapes=[
            pltpu.VMEM((2, 2, SKV, D), jnp.bfloat16),
            pltpu.SemaphoreType.DMA((2,)),
            pltpu.SemaphoreType.DMA((2,)),
        ],
        compiler_params=pltpu.CompilerParams(collective_id=0),
    )(qb16, kb16, vb16)

    out = pl.pallas_call(
        _out_body,
        out_shape=jax.ShapeDtypeStruct((SQ, D), jnp.float32),
        in_specs=[pl.BlockSpec(memory_space=pltpu.VMEM)] * 2,
        out_specs=pl.BlockSpec(memory_space=pltpu.VMEM),
    )(ctx, Wo)

    return out.reshape(1, SQ, D)


# baseline (device time: 483438 ns/iter reference)
import jax
import jax.numpy as jnp
from jax import lax
from jax.experimental import pallas as pl
from jax.experimental.pallas import tpu as pltpu

N_DEV = 4
SQ = 2048
SKV = 2048
H = 8
DH = 128
D = H * DH
BLK = 64
BPS = SKV // BLK
KV_TILE = 256
TILES = SKV // KV_TILE
SCALE = 0.08838834764831843


def _prep_body(x_ref, wq_ref, k_ref, v_ref, q_out, k_out, v_out):
    q = jnp.dot(x_ref[...], wq_ref[...], preferred_element_type=jnp.float32)
    q_out[...] = q.astype(jnp.bfloat16)
    k_out[...] = k_ref[...].astype(jnp.bfloat16)
    v_out[...] = v_ref[...].astype(jnp.bfloat16)


def _attn_body(q_ref, k_hbm, v_hbm, out_ref,
               comm_ref, mask_ref, l_ref, local_sems, send_sems, recv_sems):
    my = lax.axis_index("i")
    left = lax.rem(my + N_DEV - 1, N_DEV)
    right = lax.rem(my + 1, N_DEV)

    barrier_sem = pltpu.get_barrier_semaphore()
    for nbr in (left, right):
        pl.semaphore_signal(
            barrier_sem, inc=1,
            device_id=(nbr,), device_id_type=pl.DeviceIdType.MESH,
        )
    pl.semaphore_wait(barrier_sem, 2)

    cp_k = pltpu.make_async_copy(k_hbm, comm_ref.at[0, 0], local_sems.at[0])
    cp_v = pltpu.make_async_copy(v_hbm, comm_ref.at[0, 1], local_sems.at[1])
    cp_k.start()
    cp_v.start()

    out_ref[...] = jnp.zeros((SQ, D), jnp.float32)
    l_ref[...] = jnp.zeros((SQ, H), jnp.float32)

    row = lax.broadcasted_iota(jnp.int32, (SQ, 1), 0)
    qb = my * BPS + row // BLK

    def fill_mask(origin):
        def tile_step(t, _):
            col = lax.broadcasted_iota(jnp.int32, (1, KV_TILE), 1)
            kb = origin * BPS + (t * KV_TILE + col) // BLK
            keep = (qb == kb) | (kb == 0) | ((qb + kb) % 3 == 0)
            mask_ref[t] = keep.astype(jnp.int8)
            return 0
        lax.fori_loop(0, TILES, tile_step, 0)

    def process(slot):
        def tile_step(t, _):
            kv_rows = pl.ds(t * KV_TILE, KV_TILE)
            mf = mask_ref[t].astype(jnp.float32)
            for h in range(H):
                hd = slice(h * DH, (h + 1) * DH)
                qh = q_ref[:, hd]
                kh = comm_ref[slot, 0, kv_rows, hd]
                vh = comm_ref[slot, 1, kv_rows, hd]
                s = lax.dot_general(
                    qh, kh, (((1,), (1,)), ((), ())),
                    preferred_element_type=jnp.float32,
                )
                p = jnp.exp(s * SCALE) * mf
                l_ref[:, h:h + 1] = (
                    l_ref[:, h:h + 1] + jnp.sum(p, axis=1, keepdims=True)
                )
                pv = jnp.dot(
                    p.astype(jnp.bfloat16), vh,
                    preferred_element_type=jnp.float32,
                )
                out_ref[:, hd] = out_ref[:, hd] + pv
            return 0
        lax.fori_loop(0, TILES, tile_step, 0)

    fill_mask(my)
    cp_k.wait()
    cp_v.wait()
    process(0)

    for hop in range(N_DEV - 1):
        send_slot = hop % 2
        recv_slot = (hop + 1) % 2
        rdma = pltpu.make_async_remote_copy(
            src_ref=comm_ref.at[send_slot],
            dst_ref=comm_ref.at[recv_slot],
            send_sem=send_sems.at[send_slot],
            recv_sem=recv_sems.at[recv_slot],
            device_id=(right,),
            device_id_type=pl.DeviceIdType.MESH,
        )
        rdma.start()
        fill_mask(lax.rem(my - hop - 1 + N_DEV, N_DEV))
        rdma.wait()
        process(recv_slot)

    for h in range(H):
        hd = slice(h * DH, (h + 1) * DH)
        out_ref[:, hd] = out_ref[:, hd] / l_ref[:, h:h + 1]


def _out_body(ctx_ref, wo_ref, out_ref):
    out_ref[...] = jnp.dot(
        ctx_ref[...], wo_ref[...], preferred_element_type=jnp.float32
    )


def kernel(x, Wq, K_ext, V_ext, Wo):
    x2 = x.reshape(SQ, D)
    k2 = K_ext.reshape(SKV, D)
    v2 = V_ext.reshape(SKV, D)

    qb16, kb16, vb16 = pl.pallas_call(
        _prep_body,
        out_shape=(
            jax.ShapeDtypeStruct((SQ, D), jnp.bfloat16),
            jax.ShapeDtypeStruct((SKV, D), jnp.bfloat16),
            jax.ShapeDtypeStruct((SKV, D), jnp.bfloat16),
        ),
        in_specs=[pl.BlockSpec(memory_space=pltpu.MemorySpace.VMEM)] * 4,
        out_specs=(pl.BlockSpec(memory_space=pltpu.MemorySpace.VMEM),) * 3,
    )(x2, Wq, k2, v2)

    ctx = pl.pallas_call(
        _attn_body,
        out_shape=jax.ShapeDtypeStruct((SQ, D), jnp.float32),
        in_specs=[
            pl.BlockSpec(memory_space=pltpu.MemorySpace.VMEM),
            pl.BlockSpec(memory_space=pltpu.MemorySpace.HBM),
            pl.BlockSpec(memory_space=pltpu.MemorySpace.HBM),
        ],
        out_specs=pl.BlockSpec(memory_space=pltpu.MemorySpace.VMEM),
        scratch_shapes=[
            pltpu.VMEM((2, 2, SKV, D), jnp.bfloat16),
            pltpu.VMEM((TILES, SQ, KV_TILE), jnp.int8),
            pltpu.VMEM((SQ, H), jnp.float32),
            pltpu.SemaphoreType.DMA((2,)),
            pltpu.SemaphoreType.DMA((2,)),
            pltpu.SemaphoreType.DMA((2,)),
        ],
        compiler_params=pltpu.CompilerParams(collective_id=0),
    )(qb16, kb16, vb16)

    out = pl.pallas_call(
        _out_body,
        out_shape=jax.ShapeDtypeStruct((SQ, D), jnp.float32),
        in_specs=[pl.BlockSpec(memory_space=pltpu.MemorySpace.VMEM)] * 2,
        out_specs=pl.BlockSpec(memory_space=pltpu.MemorySpace.VMEM),
    )(ctx, Wo)

    return out.reshape(1, SQ, D)


# device time: 235853 ns/iter; 2.0497x vs baseline; 2.0497x over previous
import jax
import jax.numpy as jnp
from jax import lax
from jax.experimental import pallas as pl
from jax.experimental.pallas import tpu as pltpu

N_DEV = 4
SQ = 2048
SKV = 2048
HALF = SKV // 2
H = 8
DH = 128
D = H * DH
BLK = 64
BPS = SKV // BLK
HBLK = HALF // BLK
KV_TILE = 256
HTILES = HALF // KV_TILE
SCALE = 0.08838834764831843


def _prep_body(x_ref, wq_ref, k_ref, v_ref, q_out, k_out, v_out):
    q = jnp.dot(x_ref[...], wq_ref[...], preferred_element_type=jnp.float32)
    q_out[...] = (q * SCALE).astype(jnp.bfloat16)
    k_out[...] = k_ref[...].astype(jnp.bfloat16)
    v_out[...] = v_ref[...].astype(jnp.bfloat16)


def _attn_body(q_ref, k_hbm, v_hbm, out_ref,
               commr_ref, comml_ref, mask_ref, l_ref,
               local_sems, send_r, recv_r, send_l, recv_l):
    my = lax.axis_index("i")
    left = lax.rem(my + N_DEV - 1, N_DEV)
    right = lax.rem(my + 1, N_DEV)

    barrier_sem = pltpu.get_barrier_semaphore()
    for nbr in (left, right):
        pl.semaphore_signal(
            barrier_sem, inc=1,
            device_id=(nbr,), device_id_type=pl.DeviceIdType.MESH,
        )
    pl.semaphore_wait(barrier_sem, 2)

    cps = [
        pltpu.make_async_copy(k_hbm.at[0:HALF], commr_ref.at[0, 0],
                              local_sems.at[0]),
        pltpu.make_async_copy(v_hbm.at[0:HALF], commr_ref.at[0, 1],
                              local_sems.at[1]),
        pltpu.make_async_copy(k_hbm.at[HALF:SKV], comml_ref.at[0, 0],
                              local_sems.at[2]),
        pltpu.make_async_copy(v_hbm.at[HALF:SKV], comml_ref.at[0, 1],
                              local_sems.at[3]),
    ]
    for cp in cps:
        cp.start()

    out_ref[...] = jnp.zeros((SQ, D), jnp.float32)
    l_ref[...] = jnp.zeros((SQ, H), jnp.float32)

    row = lax.broadcasted_iota(jnp.int32, (SQ, 1), 0)
    qb = my * BPS + row // BLK

    def fill_mask(origin, tile_base, blk_off):
        def tile_step(t, _):
            col = lax.broadcasted_iota(jnp.int32, (1, KV_TILE), 1)
            kb = origin * BPS + blk_off + (t * KV_TILE + col) // BLK
            keep = (qb == kb) | (kb == 0) | ((qb + kb) % 3 == 0)
            mask_ref[tile_base + t] = keep.astype(jnp.int8)
            return 0
        lax.fori_loop(0, HTILES, tile_step, 0)

    def process(comm_ref, slot, tile_base):
        def tile_step(t, _):
            kv_rows = pl.ds(t * KV_TILE, KV_TILE)
            mf = mask_ref[tile_base + t].astype(jnp.float32)
            for h in range(H):
                hd = slice(h * DH, (h + 1) * DH)
                qh = q_ref[:, hd]
                kh = comm_ref[slot, 0, kv_rows, hd]
                vh = comm_ref[slot, 1, kv_rows, hd]
                s = lax.dot_general(
                    qh, kh, (((1,), (1,)), ((), ())),
                    preferred_element_type=jnp.float32,
                )
                p = jnp.exp(s) * mf
                l_ref[:, h:h + 1] = (
                    l_ref[:, h:h + 1] + jnp.sum(p, axis=1, keepdims=True)
                )
                pv = jnp.dot(
                    p.astype(jnp.bfloat16), vh,
                    preferred_element_type=jnp.float32,
                )
                out_ref[:, hd] = out_ref[:, hd] + pv
            return 0
        lax.fori_loop(0, HTILES, tile_step, 0)

    fill_mask(my, 0, 0)
    fill_mask(my, HTILES, HBLK)
    for cp in cps:
        cp.wait()

    for hop in range(N_DEV - 1):
        s_slot = hop % 2
        r_slot = (hop + 1) % 2
        rdma_r = pltpu.make_async_remote_copy(
            src_ref=commr_ref.at[s_slot],
            dst_ref=commr_ref.at[r_slot],
            send_sem=send_r.at[s_slot],
            recv_sem=recv_r.at[r_slot],
            device_id=(right,),
            device_id_type=pl.DeviceIdType.MESH,
        )
        rdma_l = pltpu.make_async_remote_copy(
            src_ref=comml_ref.at[s_slot],
            dst_ref=comml_ref.at[r_slot],
            send_sem=send_l.at[s_slot],
            recv_sem=recv_l.at[r_slot],
            device_id=(left,),
            device_id_type=pl.DeviceIdType.MESH,
        )
        rdma_r.start()
        rdma_l.start()
        process(commr_ref, s_slot, 0)
        process(comml_ref, s_slot, HTILES)
        fill_mask(lax.rem(my - hop - 1 + N_DEV, N_DEV), 0, 0)
        fill_mask(lax.rem(my + hop + 1, N_DEV), HTILES, HBLK)
        rdma_r.wait()
        rdma_l.wait()

    last = (N_DEV - 1) % 2
    process(commr_ref, last, 0)
    process(comml_ref, last, HTILES)

    for h in range(H):
        hd = slice(h * DH, (h + 1) * DH)
        out_ref[:, hd] = out_ref[:, hd] / l_ref[:, h:h + 1]


def _out_body(ctx_ref, wo_ref, out_ref):
    out_ref[...] = jnp.dot(
        ctx_ref[...], wo_ref[...], preferred_element_type=jnp.float32
    )


def kernel(x, Wq, K_ext, V_ext, Wo):
    x2 = x.reshape(SQ, D)
    k2 = K_ext.reshape(SKV, D)
    v2 = V_ext.reshape(SKV, D)

    qb16, kb16, vb16 = pl.pallas_call(
        _prep_body,
        out_shape=(
            jax.ShapeDtypeStruct((SQ, D), jnp.bfloat16),
            jax.ShapeDtypeStruct((SKV, D), jnp.bfloat16),
            jax.ShapeDtypeStruct((SKV, D), jnp.bfloat16),
        ),
        in_specs=[pl.BlockSpec(memory_space=pltpu.MemorySpace.VMEM)] * 4,
        out_specs=(pl.BlockSpec(memory_space=pltpu.MemorySpace.VMEM),) * 3,
    )(x2, Wq, k2, v2)

    ctx = pl.pallas_call(
        _attn_body,
        out_shape=jax.ShapeDtypeStruct((SQ, D), jnp.float32),
        in_specs=[
            pl.BlockSpec(memory_space=pltpu.MemorySpace.VMEM),
            pl.BlockSpec(memory_space=pltpu.MemorySpace.HBM),
            pl.BlockSpec(memory_space=pltpu.MemorySpace.HBM),
        ],
        out_specs=pl.BlockSpec(memory_space=pltpu.MemorySpace.VMEM),
        scratch_shapes=[
            pltpu.VMEM((2, 2, HALF, D), jnp.bfloat16),
            pltpu.VMEM((2, 2, HALF, D), jnp.bfloat16),
            pltpu.VMEM((2 * HTILES, SQ, KV_TILE), jnp.int8),
            pltpu.VMEM((SQ, H), jnp.float32),
            pltpu.SemaphoreType.DMA((4,)),
            pltpu.SemaphoreType.DMA((2,)),
            pltpu.SemaphoreType.DMA((2,)),
            pltpu.SemaphoreType.DMA((2,)),
            pltpu.SemaphoreType.DMA((2,)),
        ],
        compiler_params=pltpu.CompilerParams(collective_id=0),
    )(qb16, kb16, vb16)

    out = pl.pallas_call(
        _out_body,
        out_shape=jax.ShapeDtypeStruct((SQ, D), jnp.float32),
        in_specs=[pl.BlockSpec(memory_space=pltpu.MemorySpace.VMEM)] * 2,
        out_specs=pl.BlockSpec(memory_space=pltpu.MemorySpace.VMEM),
    )(ctx, Wo)

    return out.reshape(1, SQ, D)


# device time: 235642 ns/iter; 2.0516x vs baseline; 1.0009x over previous
import jax
import jax.numpy as jnp
from jax import lax
from jax.experimental import pallas as pl
from jax.experimental.pallas import tpu as pltpu

N_DEV = 4
SQ = 2048
SKV = 2048
HALF = SKV // 2
H = 8
DH = 128
D = H * DH
BLK = 64
BPS = SKV // BLK
HBLK = HALF // BLK
KV_TILE = 256
HTILES = HALF // KV_TILE
SCALE = 0.08838834764831843


def _prep_body(x_ref, wq_ref, k_ref, v_ref, q_out, k_out, v_out):
    q = jnp.dot(x_ref[...], wq_ref[...], preferred_element_type=jnp.float32)
    q_out[...] = (q * (SCALE * 1.4426950408889634)).astype(jnp.bfloat16)
    k_out[...] = k_ref[...].astype(jnp.bfloat16)
    v_out[...] = v_ref[...].astype(jnp.bfloat16)


def _attn_body(q_ref, k_hbm, v_hbm, out_ref,
               commr_ref, comml_ref, mask_ref, l_ref,
               local_sems, send_r, recv_r, send_l, recv_l):
    my = lax.axis_index("i")
    left = lax.rem(my + N_DEV - 1, N_DEV)
    right = lax.rem(my + 1, N_DEV)

    barrier_sem = pltpu.get_barrier_semaphore()
    for nbr in (left, right):
        pl.semaphore_signal(
            barrier_sem, inc=1,
            device_id=(nbr,), device_id_type=pl.DeviceIdType.MESH,
        )
    pl.semaphore_wait(barrier_sem, 2)

    cps = [
        pltpu.make_async_copy(k_hbm.at[0:HALF], commr_ref.at[0, 0],
                              local_sems.at[0]),
        pltpu.make_async_copy(v_hbm.at[0:HALF], commr_ref.at[0, 1],
                              local_sems.at[1]),
        pltpu.make_async_copy(k_hbm.at[HALF:SKV], comml_ref.at[0, 0],
                              local_sems.at[2]),
        pltpu.make_async_copy(v_hbm.at[HALF:SKV], comml_ref.at[0, 1],
                              local_sems.at[3]),
    ]
    for cp in cps:
        cp.start()

    out_ref[...] = jnp.zeros((SQ, D), jnp.float32)
    l_ref[...] = jnp.zeros((SQ, H), jnp.float32)

    row = lax.broadcasted_iota(jnp.int32, (SQ, 1), 0)
    qb = my * BPS + row // BLK
    qm = qb % 3

    def fill_mask(origin, tile_base, blk_off):
        def tile_step(t, _):
            col = lax.broadcasted_iota(jnp.int32, (1, KV_TILE), 1)
            kb = origin * BPS + blk_off + (t * KV_TILE + col) // BLK
            km = kb % 3
            r = qm + km
            keep = (qb == kb) | (kb == 0) | (r == 0) | (r == 3)
            mask_ref[tile_base + t] = keep.astype(jnp.int8)
            return 0
        lax.fori_loop(0, HTILES, tile_step, 0)

    def process(comm_ref, slot, tile_base):
        def tile_step(t, _):
            kv_rows = pl.ds(t * KV_TILE, KV_TILE)
            mf = mask_ref[tile_base + t].astype(jnp.float32)
            for h in range(H):
                hd = slice(h * DH, (h + 1) * DH)
                qh = q_ref[:, hd]
                kh = comm_ref[slot, 0, kv_rows, hd]
                vh = comm_ref[slot, 1, kv_rows, hd]
                s = lax.dot_general(
                    qh, kh, (((1,), (1,)), ((), ())),
                    preferred_element_type=jnp.float32,
                )
                p = jnp.exp2(s) * mf
                l_ref[:, h:h + 1] = (
                    l_ref[:, h:h + 1] + jnp.sum(p, axis=1, keepdims=True)
                )
                pv = jnp.dot(
                    p, vh.astype(jnp.float32),
                    preferred_element_type=jnp.float32,
                )
                out_ref[:, hd] = out_ref[:, hd] + pv
            return 0
        lax.fori_loop(0, HTILES, tile_step, 0)

    fill_mask(my, 0, 0)
    fill_mask(my, HTILES, HBLK)
    for cp in cps:
        cp.wait()

    for hop in range(N_DEV - 1):
        s_slot = hop % 2
        r_slot = (hop + 1) % 2
        rdma_r = pltpu.make_async_remote_copy(
            src_ref=commr_ref.at[s_slot],
            dst_ref=commr_ref.at[r_slot],
            send_sem=send_r.at[s_slot],
            recv_sem=recv_r.at[r_slot],
            device_id=(right,),
            device_id_type=pl.DeviceIdType.MESH,
        )
        rdma_l = pltpu.make_async_remote_copy(
            src_ref=comml_ref.at[s_slot],
            dst_ref=comml_ref.at[r_slot],
            send_sem=send_l.at[s_slot],
            recv_sem=recv_l.at[r_slot],
            device_id=(left,),
            device_id_type=pl.DeviceIdType.MESH,
        )
        rdma_r.start()
        rdma_l.start()
        process(commr_ref, s_slot, 0)
        process(comml_ref, s_slot, HTILES)
        fill_mask(lax.rem(my - hop - 1 + N_DEV, N_DEV), 0, 0)
        fill_mask(lax.rem(my + hop + 1, N_DEV), HTILES, HBLK)
        rdma_r.wait()
        rdma_l.wait()

    last = (N_DEV - 1) % 2
    process(commr_ref, last, 0)
    process(comml_ref, last, HTILES)

    for h in range(H):
        hd = slice(h * DH, (h + 1) * DH)
        out_ref[:, hd] = out_ref[:, hd] / l_ref[:, h:h + 1]


def _out_body(ctx_ref, wo_ref, out_ref):
    out_ref[...] = jnp.dot(
        ctx_ref[...], wo_ref[...], preferred_element_type=jnp.float32
    )


def kernel(x, Wq, K_ext, V_ext, Wo):
    x2 = x.reshape(SQ, D)
    k2 = K_ext.reshape(SKV, D)
    v2 = V_ext.reshape(SKV, D)

    qb16, kb16, vb16 = pl.pallas_call(
        _prep_body,
        out_shape=(
            jax.ShapeDtypeStruct((SQ, D), jnp.bfloat16),
            jax.ShapeDtypeStruct((SKV, D), jnp.bfloat16),
            jax.ShapeDtypeStruct((SKV, D), jnp.bfloat16),
        ),
        in_specs=[pl.BlockSpec(memory_space=pltpu.MemorySpace.VMEM)] * 4,
        out_specs=(pl.BlockSpec(memory_space=pltpu.MemorySpace.VMEM),) * 3,
    )(x2, Wq, k2, v2)

    ctx = pl.pallas_call(
        _attn_body,
        out_shape=jax.ShapeDtypeStruct((SQ, D), jnp.float32),
        in_specs=[
            pl.BlockSpec(memory_space=pltpu.MemorySpace.VMEM),
            pl.BlockSpec(memory_space=pltpu.MemorySpace.HBM),
            pl.BlockSpec(memory_space=pltpu.MemorySpace.HBM),
        ],
        out_specs=pl.BlockSpec(memory_space=pltpu.MemorySpace.VMEM),
        scratch_shapes=[
            pltpu.VMEM((2, 2, HALF, D), jnp.bfloat16),
            pltpu.VMEM((2, 2, HALF, D), jnp.bfloat16),
            pltpu.VMEM((2 * HTILES, SQ, KV_TILE), jnp.int8),
            pltpu.VMEM((SQ, H), jnp.float32),
            pltpu.SemaphoreType.DMA((4,)),
            pltpu.SemaphoreType.DMA((2,)),
            pltpu.SemaphoreType.DMA((2,)),
            pltpu.SemaphoreType.DMA((2,)),
            pltpu.SemaphoreType.DMA((2,)),
        ],
        compiler_params=pltpu.CompilerParams(collective_id=0),
    )(qb16, kb16, vb16)

    out = pl.pallas_call(
        _out_body,
        out_shape=jax.ShapeDtypeStruct((SQ, D), jnp.float32),
        in_specs=[pl.BlockSpec(memory_space=pltpu.MemorySpace.VMEM)] * 2,
        out_specs=pl.BlockSpec(memory_space=pltpu.MemorySpace.VMEM),
    )(ctx, Wo)

    return out.reshape(1, SQ, D)
